# Initial kernel scaffold; baseline (speedup 1.0000x reference)
import jax
import jax.numpy as jnp
from jax.experimental import pallas as pl


def kernel(x, edge_index, W1, b1, gamma, beta, W2, b2):
    raise NotImplementedError("write your pallas kernel here")


# speedup vs baseline: 12.1431x; 12.1431x over previous
import functools

import jax
import jax.numpy as jnp
from jax import lax
from jax.experimental import pallas as pl
from jax.experimental.pallas import tpu as pltpu
from jax.experimental.pallas import tpu_sc as plsc

_N = 10000
_E = 320000
_D = 128
_H = 16
_C = 64
_EPS = 1e-5

_NP = 10240
_ROWS = _NP // 16
_NW = 32
_CH = 128
_NCHUNK = 80
_EW = _CH * _NCHUNK

_SELU_ALPHA = 1.6732632423543772
_SELU_SCALE = 1.0507009873554805


def _sc_scatter_partials(h_pad, src3, dst3, zeros_pad):
    mesh = plsc.VectorSubcoreMesh(core_axis_name="c", subcore_axis_name="s")

    @functools.partial(
        pl.kernel,
        out_type=jax.ShapeDtypeStruct((2, _NP, _H), jnp.float32),
        mesh=mesh,
        scratch_types=[
            pltpu.VMEM((_NCHUNK, _CH), jnp.int32),
            pltpu.VMEM((_NCHUNK, _CH), jnp.int32),
            pltpu.VMEM((2, _CH, _H), jnp.float32),
            pltpu.VMEM_SHARED((_NP, _H), jnp.float32),
            pltpu.SemaphoreType.DMA,
            pltpu.SemaphoreType.DMA,
        ],
        compiler_params=pltpu.CompilerParams(use_tc_tiling_on_sc=False),
    )
    def k(h_hbm, src_hbm, dst_hbm, z_hbm, out_hbm, src_v, dst_v, rows_v,
          acc, gsem, ssem):
        c = lax.axis_index("c")
        s = lax.axis_index("s")
        wid = s * 2 + c
        row0 = s * _ROWS
        pltpu.sync_copy(z_hbm.at[pl.ds(row0, _ROWS)],
                        acc.at[pl.ds(row0, _ROWS)])
        pltpu.sync_copy(src_hbm.at[wid], src_v)
        pltpu.sync_copy(dst_hbm.at[wid], dst_v)
        plsc.subcore_barrier()

        g0 = pltpu.async_copy(h_hbm.at[src_v.at[0]], rows_v.at[0], gsem)
        g0.wait()

        def body(j, _):
            slot = lax.rem(j, 2)
            nxt = lax.rem(j + 1, 2)

            @pl.when(j + 1 < _NCHUNK)
            def _():
                pltpu.async_copy(h_hbm.at[src_v.at[j + 1]], rows_v.at[nxt],
                                 gsem)

            pltpu.async_copy(rows_v.at[slot], acc.at[dst_v.at[j]], ssem,
                             add=True).wait()

            @pl.when(j + 1 < _NCHUNK)
            def _():
                pltpu.make_async_copy(h_hbm.at[src_v.at[j + 1]],
                                      rows_v.at[nxt], gsem).wait()
            return 0

        lax.fori_loop(0, _NCHUNK, body, 0)
        plsc.subcore_barrier()
        pltpu.sync_copy(acc.at[pl.ds(row0, _ROWS)],
                        out_hbm.at[c, pl.ds(row0, _ROWS)])

    return k(h_pad, src3, dst3, zeros_pad)


def _tc_matmul1(x_pad, W1):
    def body(x_ref, w_ref, o_ref):
        o_ref[...] = jnp.dot(x_ref[...], w_ref[...],
                             preferred_element_type=jnp.float32)

    return pl.pallas_call(
        body,
        out_shape=jax.ShapeDtypeStruct((_NP, _H), jnp.float32),
        grid=(8,),
        in_specs=[pl.BlockSpec((_NP // 8, _D), lambda i: (i, 0)),
                  pl.BlockSpec((_D, _H), lambda i: (0, 0))],
        out_specs=pl.BlockSpec((_NP // 8, _H), lambda i: (i, 0)),
    )(x_pad, W1)


def _tc_combine(h, p):
    hv = h.reshape(_NP * _H // 128, 128)
    pv = p.reshape(2, _NP * _H // 128, 128)

    def body(h_ref, p_ref, o_ref):
        o_ref[...] = h_ref[...] + p_ref[0] + p_ref[1]

    out = pl.pallas_call(
        body,
        out_shape=jax.ShapeDtypeStruct((_NP * _H // 128, 128), jnp.float32),
        grid=(4,),
        in_specs=[pl.BlockSpec((_NP * _H // 512, 128), lambda i: (i, 0)),
                  pl.BlockSpec((2, _NP * _H // 512, 128), lambda i: (0, i, 0))],
        out_specs=pl.BlockSpec((_NP * _H // 512, 128), lambda i: (i, 0)),
    )(hv, pv)
    return out.reshape(_NP, _H)


def _tc_bn_selu(h1, p2, b1, gamma, beta):
    def body(h_ref, p_ref, b_ref, g_ref, be_ref, o_ref):
        h2 = h_ref[...] + p_ref[0] + p_ref[1] + b_ref[...]
        rows = lax.broadcasted_iota(jnp.int32, (_NP, _H), 0)
        mask = rows < _N
        hm = jnp.where(mask, h2, 0.0)
        mean = jnp.sum(hm, axis=0, keepdims=True) / _N
        d = jnp.where(mask, h2 - mean, 0.0)
        var = jnp.sum(d * d, axis=0, keepdims=True) / _N
        z = (h2 - mean) * lax.rsqrt(var + _EPS) * g_ref[...] + be_ref[...]
        o_ref[...] = _SELU_SCALE * jnp.where(
            z > 0, z, _SELU_ALPHA * (jnp.exp(z) - 1.0))

    return pl.pallas_call(
        body,
        out_shape=jax.ShapeDtypeStruct((_NP, _H), jnp.float32),
        in_specs=[pl.BlockSpec((_NP, _H), lambda: (0, 0)),
                  pl.BlockSpec((2, _NP, _H), lambda: (0, 0, 0)),
                  pl.BlockSpec((1, _H), lambda: (0, 0)),
                  pl.BlockSpec((1, _H), lambda: (0, 0)),
                  pl.BlockSpec((1, _H), lambda: (0, 0))],
        out_specs=pl.BlockSpec((_NP, _H), lambda: (0, 0)),
    )(h1, p2, b1.reshape(1, _H), gamma.reshape(1, _H), beta.reshape(1, _H))


def _tc_head(s, p3, W2, b2):
    blk = 2000

    def body(s_ref, p_ref, w_ref, b_ref, o_ref):
        t = s_ref[...] + p_ref[0] + p_ref[1]
        z = jnp.dot(t, w_ref[...],
                    preferred_element_type=jnp.float32) + b_ref[...]
        m = jnp.max(z, axis=1, keepdims=True)
        e = jnp.exp(z - m)
        lse = jnp.log(jnp.sum(e, axis=1, keepdims=True)) + m
        o_ref[...] = z - lse

    return pl.pallas_call(
        body,
        out_shape=jax.ShapeDtypeStruct((_N, _C), jnp.float32),
        grid=(_N // blk,),
        in_specs=[pl.BlockSpec((blk, _H), lambda i: (i, 0)),
                  pl.BlockSpec((2, blk, _H), lambda i: (0, i, 0)),
                  pl.BlockSpec((_H, _C), lambda i: (0, 0)),
                  pl.BlockSpec((1, _C), lambda i: (0, 0))],
        out_specs=pl.BlockSpec((blk, _C), lambda i: (i, 0)),
    )(s, p3, W2, b2.reshape(1, _C))


@jax.jit
def kernel(x, edge_index, W1, b1, gamma, beta, W2, b2):
    src = edge_index[0]
    dst = edge_index[1]
    pad = _NW * _EW - _E
    src3 = jnp.concatenate(
        [src, jnp.full((pad,), _NP - 1, jnp.int32)]).reshape(_NW, _NCHUNK, _CH)
    dst3 = jnp.concatenate(
        [dst, jnp.full((pad,), _N, jnp.int32)]).reshape(_NW, _NCHUNK, _CH)
    zeros_pad = jnp.zeros((_NP, _H), jnp.float32)

    x_pad = jnp.pad(x, ((0, _NP - _N), (0, 0)))
    h0 = _tc_matmul1(x_pad, W1)
    p1 = _sc_scatter_partials(h0, src3, dst3, zeros_pad)
    h1 = _tc_combine(h0, p1)
    p2 = _sc_scatter_partials(h1, src3, dst3, zeros_pad)
    s = _tc_bn_selu(h1, p2, b1, gamma, beta)
    p3 = _sc_scatter_partials(s, src3, dst3, zeros_pad)
    return _tc_head(s, p3, W2, b2)


# speedup vs baseline: 15.5591x; 1.2813x over previous
import functools

import jax
import jax.numpy as jnp
from jax import lax
from jax.experimental import pallas as pl
from jax.experimental.pallas import tpu as pltpu
from jax.experimental.pallas import tpu_sc as plsc

_N = 10000
_E = 320000
_D = 128
_H = 16
_C = 64
_EPS = 1e-5

_NP = 10240
_ROWS = _NP // 16
_NW = 32
_SUP = 2560
_NSUP = 4
_EW = _SUP * _NSUP

_SELU_ALPHA = 1.6732632423543772
_SELU_SCALE = 1.0507009873554805


def _sc_scatter_partials(h_pad, src3, dst3, zeros_pad):
    mesh = plsc.VectorSubcoreMesh(core_axis_name="c", subcore_axis_name="s")

    @functools.partial(
        pl.kernel,
        out_type=jax.ShapeDtypeStruct((2, _NP, _H), jnp.float32),
        mesh=mesh,
        scratch_types=[
            pltpu.VMEM((_NSUP, _SUP), jnp.int32),
            pltpu.VMEM((_NSUP, _SUP), jnp.int32),
            pltpu.VMEM((2, _SUP, _H), jnp.float32),
            pltpu.VMEM_SHARED((_NP, _H), jnp.float32),
            pltpu.SemaphoreType.DMA,
            pltpu.SemaphoreType.DMA,
        ],
        compiler_params=pltpu.CompilerParams(use_tc_tiling_on_sc=False),
    )
    def k(h_hbm, src_hbm, dst_hbm, z_hbm, out_hbm, src_v, dst_v, rows_v,
          acc, gsem, ssem):
        c = lax.axis_index("c")
        s = lax.axis_index("s")
        wid = s * 2 + c
        row0 = s * _ROWS
        pltpu.sync_copy(z_hbm.at[pl.ds(row0, _ROWS)],
                        acc.at[pl.ds(row0, _ROWS)])
        pltpu.sync_copy(src_hbm.at[wid], src_v)
        pltpu.sync_copy(dst_hbm.at[wid], dst_v)
        plsc.subcore_barrier()

        g0 = pltpu.async_copy(h_hbm.at[src_v.at[0]], rows_v.at[0], gsem)
        g0.wait()

        def body(j, _):
            slot = lax.rem(j, 2)
            nxt = lax.rem(j + 1, 2)

            @pl.when(j + 1 < _NSUP)
            def _():
                pltpu.async_copy(h_hbm.at[src_v.at[j + 1]], rows_v.at[nxt],
                                 gsem)

            pltpu.async_copy(rows_v.at[slot], acc.at[dst_v.at[j]], ssem,
                             add=True).wait()

            @pl.when(j + 1 < _NSUP)
            def _():
                pltpu.make_async_copy(h_hbm.at[src_v.at[j + 1]],
                                      rows_v.at[nxt], gsem).wait()
            return 0

        lax.fori_loop(0, _NSUP, body, 0)
        plsc.subcore_barrier()
        pltpu.sync_copy(acc.at[pl.ds(row0, _ROWS)],
                        out_hbm.at[c, pl.ds(row0, _ROWS)])

    return k(h_pad, src3, dst3, zeros_pad)


def _tc_matmul1(x_pad, W1):
    def body(x_ref, w_ref, o_ref):
        o_ref[...] = jnp.dot(x_ref[...], w_ref[...],
                             preferred_element_type=jnp.float32)

    return pl.pallas_call(
        body,
        out_shape=jax.ShapeDtypeStruct((_NP, _H), jnp.float32),
        grid=(8,),
        in_specs=[pl.BlockSpec((_NP // 8, _D), lambda i: (i, 0)),
                  pl.BlockSpec((_D, _H), lambda i: (0, 0))],
        out_specs=pl.BlockSpec((_NP // 8, _H), lambda i: (i, 0)),
    )(x_pad, W1)


def _tc_combine(h, p):
    hv = h.reshape(_NP * _H // 128, 128)
    pv = p.reshape(2, _NP * _H // 128, 128)

    def body(h_ref, p_ref, o_ref):
        o_ref[...] = h_ref[...] + p_ref[0] + p_ref[1]

    out = pl.pallas_call(
        body,
        out_shape=jax.ShapeDtypeStruct((_NP * _H // 128, 128), jnp.float32),
        grid=(4,),
        in_specs=[pl.BlockSpec((_NP * _H // 512, 128), lambda i: (i, 0)),
                  pl.BlockSpec((2, _NP * _H // 512, 128), lambda i: (0, i, 0))],
        out_specs=pl.BlockSpec((_NP * _H // 512, 128), lambda i: (i, 0)),
    )(hv, pv)
    return out.reshape(_NP, _H)


def _tc_bn_selu(h1, p2, b1, gamma, beta):
    def body(h_ref, p_ref, b_ref, g_ref, be_ref, o_ref):
        h2 = h_ref[...] + p_ref[0] + p_ref[1] + b_ref[...]
        rows = lax.broadcasted_iota(jnp.int32, (_NP, _H), 0)
        mask = rows < _N
        hm = jnp.where(mask, h2, 0.0)
        mean = jnp.sum(hm, axis=0, keepdims=True) / _N
        d = jnp.where(mask, h2 - mean, 0.0)
        var = jnp.sum(d * d, axis=0, keepdims=True) / _N
        z = (h2 - mean) * lax.rsqrt(var + _EPS) * g_ref[...] + be_ref[...]
        o_ref[...] = _SELU_SCALE * jnp.where(
            z > 0, z, _SELU_ALPHA * (jnp.exp(z) - 1.0))

    return pl.pallas_call(
        body,
        out_shape=jax.ShapeDtypeStruct((_NP, _H), jnp.float32),
        in_specs=[pl.BlockSpec((_NP, _H), lambda: (0, 0)),
                  pl.BlockSpec((2, _NP, _H), lambda: (0, 0, 0)),
                  pl.BlockSpec((1, _H), lambda: (0, 0)),
                  pl.BlockSpec((1, _H), lambda: (0, 0)),
                  pl.BlockSpec((1, _H), lambda: (0, 0))],
        out_specs=pl.BlockSpec((_NP, _H), lambda: (0, 0)),
    )(h1, p2, b1.reshape(1, _H), gamma.reshape(1, _H), beta.reshape(1, _H))


def _tc_head(s, p3, W2, b2):
    blk = 2000

    def body(s_ref, p_ref, w_ref, b_ref, o_ref):
        t = s_ref[...] + p_ref[0] + p_ref[1]
        z = jnp.dot(t, w_ref[...],
                    preferred_element_type=jnp.float32) + b_ref[...]
        m = jnp.max(z, axis=1, keepdims=True)
        e = jnp.exp(z - m)
        lse = jnp.log(jnp.sum(e, axis=1, keepdims=True)) + m
        o_ref[...] = z - lse

    return pl.pallas_call(
        body,
        out_shape=jax.ShapeDtypeStruct((_N, _C), jnp.float32),
        grid=(_N // blk,),
        in_specs=[pl.BlockSpec((blk, _H), lambda i: (i, 0)),
                  pl.BlockSpec((2, blk, _H), lambda i: (0, i, 0)),
                  pl.BlockSpec((_H, _C), lambda i: (0, 0)),
                  pl.BlockSpec((1, _C), lambda i: (0, 0))],
        out_specs=pl.BlockSpec((blk, _C), lambda i: (i, 0)),
    )(s, p3, W2, b2.reshape(1, _C))


@jax.jit
def kernel(x, edge_index, W1, b1, gamma, beta, W2, b2):
    src = edge_index[0]
    dst = edge_index[1]
    pad = _NW * _EW - _E
    src3 = jnp.concatenate(
        [src, jnp.full((pad,), _NP - 1, jnp.int32)]).reshape(_NW, _NSUP, _SUP)
    dst3 = jnp.concatenate(
        [dst, jnp.full((pad,), _N, jnp.int32)]).reshape(_NW, _NSUP, _SUP)
    zeros_pad = jnp.zeros((_NP, _H), jnp.float32)

    x_pad = jnp.pad(x, ((0, _NP - _N), (0, 0)))
    h0 = _tc_matmul1(x_pad, W1)
    p1 = _sc_scatter_partials(h0, src3, dst3, zeros_pad)
    h1 = _tc_combine(h0, p1)
    p2 = _sc_scatter_partials(h1, src3, dst3, zeros_pad)
    s = _tc_bn_selu(h1, p2, b1, gamma, beta)
    p3 = _sc_scatter_partials(s, src3, dst3, zeros_pad)
    return _tc_head(s, p3, W2, b2)


# speedup vs baseline: 23.0202x; 1.4795x over previous
import functools

import jax
import jax.numpy as jnp
from jax import lax
from jax.experimental import pallas as pl
from jax.experimental.pallas import tpu as pltpu
from jax.experimental.pallas import tpu_sc as plsc

_N = 10000
_E = 320000
_D = 128
_H = 16
_C = 64
_EPS = 1e-5

_NP = 10240
_ROWS = _NP // 16
_NW = 32
_SUP = 2560
_NSUP = 4
_EW = _SUP * _NSUP

_SELU_ALPHA = 1.6732632423543772
_SELU_SCALE = 1.0507009873554805


def _sc_scatter_partials(h_pad, src3, dst3, zeros_pad):
    mesh = plsc.VectorSubcoreMesh(core_axis_name="c", subcore_axis_name="s")

    @functools.partial(
        pl.kernel,
        out_type=jax.ShapeDtypeStruct((2, _NP, _H), jnp.float32),
        mesh=mesh,
        scratch_types=[
            pltpu.VMEM((_NSUP, _SUP), jnp.int32),
            pltpu.VMEM((_NSUP, _SUP), jnp.int32),
            pltpu.VMEM((2, _SUP, _H), jnp.float32),
            pltpu.VMEM_SHARED((_NP, _H), jnp.float32),
            pltpu.VMEM_SHARED((_NP, _H), jnp.float32),
            pltpu.SemaphoreType.DMA,
            pltpu.SemaphoreType.DMA,
        ],
        compiler_params=pltpu.CompilerParams(use_tc_tiling_on_sc=False),
    )
    def k(h_hbm, src_hbm, dst_hbm, z_hbm, out_hbm, src_v, dst_v, rows_v,
          acc, tbl, gsem, ssem):
        c = lax.axis_index("c")
        s = lax.axis_index("s")
        wid = s * 2 + c
        row0 = s * _ROWS
        pltpu.sync_copy(z_hbm.at[pl.ds(row0, _ROWS)],
                        acc.at[pl.ds(row0, _ROWS)])
        pltpu.sync_copy(h_hbm.at[pl.ds(row0, _ROWS)],
                        tbl.at[pl.ds(row0, _ROWS)])
        pltpu.sync_copy(src_hbm.at[wid], src_v)
        pltpu.sync_copy(dst_hbm.at[wid], dst_v)
        plsc.subcore_barrier()

        g0 = pltpu.async_copy(tbl.at[src_v.at[0]], rows_v.at[0], gsem)
        g0.wait()

        def body(j, _):
            slot = lax.rem(j, 2)
            nxt = lax.rem(j + 1, 2)

            @pl.when(j + 1 < _NSUP)
            def _():
                pltpu.async_copy(tbl.at[src_v.at[j + 1]], rows_v.at[nxt],
                                 gsem)

            pltpu.async_copy(rows_v.at[slot], acc.at[dst_v.at[j]], ssem,
                             add=True).wait()

            @pl.when(j + 1 < _NSUP)
            def _():
                pltpu.make_async_copy(tbl.at[src_v.at[j + 1]],
                                      rows_v.at[nxt], gsem).wait()
            return 0

        lax.fori_loop(0, _NSUP, body, 0)
        plsc.subcore_barrier()
        pltpu.sync_copy(acc.at[pl.ds(row0, _ROWS)],
                        out_hbm.at[c, pl.ds(row0, _ROWS)])

    return k(h_pad, src3, dst3, zeros_pad)


def _tc_matmul1(x_pad, W1):
    def body(x_ref, w_ref, o_ref):
        o_ref[...] = jnp.dot(x_ref[...], w_ref[...],
                             preferred_element_type=jnp.float32)

    return pl.pallas_call(
        body,
        out_shape=jax.ShapeDtypeStruct((_NP, _H), jnp.float32),
        grid=(8,),
        in_specs=[pl.BlockSpec((_NP // 8, _D), lambda i: (i, 0)),
                  pl.BlockSpec((_D, _H), lambda i: (0, 0))],
        out_specs=pl.BlockSpec((_NP // 8, _H), lambda i: (i, 0)),
    )(x_pad, W1)


def _tc_combine(h, p):
    hv = h.reshape(_NP * _H // 128, 128)
    pv = p.reshape(2, _NP * _H // 128, 128)

    def body(h_ref, p_ref, o_ref):
        o_ref[...] = h_ref[...] + p_ref[0] + p_ref[1]

    out = pl.pallas_call(
        body,
        out_shape=jax.ShapeDtypeStruct((_NP * _H // 128, 128), jnp.float32),
        grid=(4,),
        in_specs=[pl.BlockSpec((_NP * _H // 512, 128), lambda i: (i, 0)),
                  pl.BlockSpec((2, _NP * _H // 512, 128), lambda i: (0, i, 0))],
        out_specs=pl.BlockSpec((_NP * _H // 512, 128), lambda i: (i, 0)),
    )(hv, pv)
    return out.reshape(_NP, _H)


def _tc_bn_selu(h1, p2, b1, gamma, beta):
    def body(h_ref, p_ref, b_ref, g_ref, be_ref, o_ref):
        h2 = h_ref[...] + p_ref[0] + p_ref[1] + b_ref[...]
        rows = lax.broadcasted_iota(jnp.int32, (_NP, _H), 0)
        mask = rows < _N
        hm = jnp.where(mask, h2, 0.0)
        mean = jnp.sum(hm, axis=0, keepdims=True) / _N
        d = jnp.where(mask, h2 - mean, 0.0)
        var = jnp.sum(d * d, axis=0, keepdims=True) / _N
        z = (h2 - mean) * lax.rsqrt(var + _EPS) * g_ref[...] + be_ref[...]
        o_ref[...] = _SELU_SCALE * jnp.where(
            z > 0, z, _SELU_ALPHA * (jnp.exp(z) - 1.0))

    return pl.pallas_call(
        body,
        out_shape=jax.ShapeDtypeStruct((_NP, _H), jnp.float32),
        in_specs=[pl.BlockSpec((_NP, _H), lambda: (0, 0)),
                  pl.BlockSpec((2, _NP, _H), lambda: (0, 0, 0)),
                  pl.BlockSpec((1, _H), lambda: (0, 0)),
                  pl.BlockSpec((1, _H), lambda: (0, 0)),
                  pl.BlockSpec((1, _H), lambda: (0, 0))],
        out_specs=pl.BlockSpec((_NP, _H), lambda: (0, 0)),
    )(h1, p2, b1.reshape(1, _H), gamma.reshape(1, _H), beta.reshape(1, _H))


def _tc_head(s, p3, W2, b2):
    blk = 2000

    def body(s_ref, p_ref, w_ref, b_ref, o_ref):
        t = s_ref[...] + p_ref[0] + p_ref[1]
        z = jnp.dot(t, w_ref[...],
                    preferred_element_type=jnp.float32) + b_ref[...]
        m = jnp.max(z, axis=1, keepdims=True)
        e = jnp.exp(z - m)
        lse = jnp.log(jnp.sum(e, axis=1, keepdims=True)) + m
        o_ref[...] = z - lse

    return pl.pallas_call(
        body,
        out_shape=jax.ShapeDtypeStruct((_N, _C), jnp.float32),
        grid=(_N // blk,),
        in_specs=[pl.BlockSpec((blk, _H), lambda i: (i, 0)),
                  pl.BlockSpec((2, blk, _H), lambda i: (0, i, 0)),
                  pl.BlockSpec((_H, _C), lambda i: (0, 0)),
                  pl.BlockSpec((1, _C), lambda i: (0, 0))],
        out_specs=pl.BlockSpec((blk, _C), lambda i: (i, 0)),
    )(s, p3, W2, b2.reshape(1, _C))


@jax.jit
def kernel(x, edge_index, W1, b1, gamma, beta, W2, b2):
    src = edge_index[0]
    dst = edge_index[1]
    pad = _NW * _EW - _E
    src3 = jnp.concatenate(
        [src, jnp.full((pad,), _NP - 1, jnp.int32)]).reshape(_NW, _NSUP, _SUP)
    dst3 = jnp.concatenate(
        [dst, jnp.full((pad,), _N, jnp.int32)]).reshape(_NW, _NSUP, _SUP)
    zeros_pad = jnp.zeros((_NP, _H), jnp.float32)

    x_pad = jnp.pad(x, ((0, _NP - _N), (0, 0)))
    h0 = _tc_matmul1(x_pad, W1)
    p1 = _sc_scatter_partials(h0, src3, dst3, zeros_pad)
    h1 = _tc_combine(h0, p1)
    p2 = _sc_scatter_partials(h1, src3, dst3, zeros_pad)
    s = _tc_bn_selu(h1, p2, b1, gamma, beta)
    p3 = _sc_scatter_partials(s, src3, dst3, zeros_pad)
    return _tc_head(s, p3, W2, b2)
